# Initial kernel scaffold; baseline (speedup 1.0000x reference)
import functools

import jax
import jax.numpy as jnp
from jax import lax
from jax.experimental import pallas as pl
from jax.experimental.pallas import tpu as pltpu
from jax.experimental.pallas import tpu_sc as plsc

N = 10000
E = 320000
D_IN = 128
DH = 16
NC = 2
NS = 16
NW = NC * NS
CHUNK = 128
CH_T = 79
EP = NW * CH_T * CHUNK
NACC = 10016
ZR = NACC // NS

_MESH = plsc.VectorSubcoreMesh(core_axis_name="c", subcore_axis_name="s")


def _sc_hist(dst2d):
    @functools.partial(
        pl.kernel,
        out_type=jax.ShapeDtypeStruct((NC * NACC, DH), jnp.float32),
        mesh=_MESH,
        scratch_types=[
            pltpu.VMEM((CH_T, CHUNK), jnp.int32),
            pltpu.VMEM((CHUNK, DH), jnp.float32),
            pltpu.VMEM((ZR, DH), jnp.float32),
            pltpu.VMEM_SHARED((NACC, DH), jnp.float32),
        ],
    )
    def k(dst_hbm, out_hbm, dstv, ones_v, zbuf, acc):
        cid = lax.axis_index("c")
        sid = lax.axis_index("s")
        wid = sid * NC + cid

        @pl.loop(0, ZR)
        def _(i):
            zbuf.at[pl.ds(i, 1), pl.ds(0, DH)][...] = jnp.zeros((1, DH), jnp.float32)

        @pl.loop(0, CHUNK)
        def _(i):
            ones_v.at[pl.ds(i, 1), pl.ds(0, DH)][...] = jnp.ones((1, DH), jnp.float32)

        pltpu.sync_copy(dst_hbm.at[pl.ds(wid * CH_T, CH_T)], dstv)
        pltpu.sync_copy(zbuf, acc.at[pl.ds(sid * ZR, ZR)])
        plsc.subcore_barrier()

        @pl.loop(0, CH_T)
        def _(j):
            pltpu.sync_copy(ones_v, acc.at[dstv.at[j]], add=True)

        plsc.subcore_barrier()
        pltpu.sync_copy(
            acc.at[pl.ds(sid * ZR, ZR)],
            out_hbm.at[pl.ds(cid * NACC + sid * ZR, ZR)],
        )

    return k(dst2d)


def _sc_agg(src2d, dst2d, y):
    @functools.partial(
        pl.kernel,
        out_type=jax.ShapeDtypeStruct((NC * NACC, DH), jnp.float32),
        mesh=_MESH,
        scratch_types=[
            pltpu.VMEM((CH_T, CHUNK), jnp.int32),
            pltpu.VMEM((CH_T, CHUNK), jnp.int32),
            pltpu.VMEM((CHUNK, DH), jnp.float32),
            pltpu.VMEM((ZR, DH), jnp.float32),
            pltpu.VMEM_SHARED((NACC, DH), jnp.float32),
        ],
    )
    def k(src_hbm, dst_hbm, y_hbm, out_hbm, srcv, dstv, rows, zbuf, acc):
        cid = lax.axis_index("c")
        sid = lax.axis_index("s")
        wid = sid * NC + cid

        @pl.loop(0, ZR)
        def _(i):
            zbuf.at[pl.ds(i, 1), pl.ds(0, DH)][...] = jnp.zeros((1, DH), jnp.float32)

        pltpu.sync_copy(src_hbm.at[pl.ds(wid * CH_T, CH_T)], srcv)
        pltpu.sync_copy(dst_hbm.at[pl.ds(wid * CH_T, CH_T)], dstv)
        pltpu.sync_copy(zbuf, acc.at[pl.ds(sid * ZR, ZR)])
        plsc.subcore_barrier()

        @pl.loop(0, CH_T)
        def _(j):
            pltpu.sync_copy(y_hbm.at[srcv.at[j]], rows)
            pltpu.sync_copy(rows, acc.at[dstv.at[j]], add=True)

        plsc.subcore_barrier()
        pltpu.sync_copy(
            acc.at[pl.ds(sid * ZR, ZR)],
            out_hbm.at[pl.ds(cid * NACC + sid * ZR, ZR)],
        )

    return k(src2d, dst2d, y)


def _tc_first(xp, W1, histp):
    def body(h_ref, x_ref, w_ref, y_ref, d_ref):
        hp = h_ref[...]
        cnt = hp[:NACC, 0:1] + hp[NACC:, 0:1]
        dinv = lax.rsqrt(cnt + 1.0)
        xw = jnp.dot(x_ref[...], w_ref[...],
                     preferred_element_type=jnp.float32,
                     precision=lax.Precision.HIGHEST)
        y_ref[...] = xw * dinv
        d_ref[...] = dinv

    return pl.pallas_call(
        body,
        out_shape=[
            jax.ShapeDtypeStruct((NACC, DH), jnp.float32),
            jax.ShapeDtypeStruct((NACC, 1), jnp.float32),
        ],
    )(histp, xp, W1)


def _tc_mid(aggp, y1, dinv, W2, b1):
    def body(a_ref, y_ref, d_ref, w_ref, b_ref, o_ref):
        ap = a_ref[...]
        z = (ap[:NACC] + ap[NACC:] + y_ref[...]) * d_ref[...] + b_ref[...]
        h = jnp.maximum(z, 0.0)
        o_ref[...] = jnp.dot(h, w_ref[...],
                             preferred_element_type=jnp.float32,
                             precision=lax.Precision.HIGHEST) * d_ref[...]

    return pl.pallas_call(
        body,
        out_shape=jax.ShapeDtypeStruct((NACC, DH), jnp.float32),
    )(aggp, y1, dinv, W2, b1)


def _tc_last(aggp, y2, dinv, b2):
    def body(a_ref, y_ref, d_ref, b_ref, o_ref):
        ap = a_ref[...]
        z = (ap[:NACC] + ap[NACC:] + y_ref[...]) * d_ref[...] + b_ref[...]
        m = jnp.max(z, axis=1, keepdims=True)
        s = z - m
        lse = jnp.log(jnp.sum(jnp.exp(s), axis=1, keepdims=True))
        o_ref[...] = s - lse

    return pl.pallas_call(
        body,
        out_shape=jax.ShapeDtypeStruct((NACC, DH), jnp.float32),
    )(aggp, y2, dinv, b2)


def kernel(x, edge_index, W1, b1, W2, b2):
    src = edge_index[0]
    dst = edge_index[1]
    pad = jnp.full((EP - E,), N, jnp.int32)
    src2d = jnp.concatenate([src, pad]).reshape(NW * CH_T, CHUNK)
    dst2d = jnp.concatenate([dst, pad]).reshape(NW * CH_T, CHUNK)
    xp = jnp.zeros((NACC, D_IN), jnp.float32).at[:N].set(x)

    histp = _sc_hist(dst2d)
    y1, dinv = _tc_first(xp, W1, histp)
    agg1 = _sc_agg(src2d, dst2d, y1)
    y2 = _tc_mid(agg1, y1, dinv, W2, b1.reshape(1, DH))
    agg2 = _sc_agg(src2d, dst2d, y2)
    out = _tc_last(agg2, y2, dinv, b2.reshape(1, DH))
    return out[:N]


# speedup vs baseline: 27.7965x; 27.7965x over previous
import functools

import jax
import jax.numpy as jnp
from jax import lax
from jax.experimental import pallas as pl
from jax.experimental.pallas import tpu as pltpu
from jax.experimental.pallas import tpu_sc as plsc

N = 10000
E = 320000
D_IN = 128
DH = 16
NC = 2
NS = 16
NW = NC * NS
CHUNK = 128
CH_T = 80
EP = NW * CH_T * CHUNK
NACC = 10112
ZR = NACC // NS

_MESH = plsc.VectorSubcoreMesh(core_axis_name="c", subcore_axis_name="s")
_SC_PARAMS = pltpu.CompilerParams(use_tc_tiling_on_sc=False)


def _sc_hist(dst2d):
    @functools.partial(
        pl.kernel,
        out_type=jax.ShapeDtypeStruct((NC * NACC, DH), jnp.float32),
        mesh=_MESH,
        compiler_params=_SC_PARAMS,
        scratch_types=[
            pltpu.VMEM((CH_T, CHUNK), jnp.int32),
            pltpu.VMEM((CHUNK, DH), jnp.float32),
            pltpu.VMEM((ZR, DH), jnp.float32),
            pltpu.VMEM_SHARED((NACC, DH), jnp.float32),
        ],
    )
    def k(dst_hbm, out_hbm, dstv, ones_v, zbuf, acc):
        cid = lax.axis_index("c")
        sid = lax.axis_index("s")
        wid = sid * NC + cid

        @pl.loop(0, ZR)
        def _(i):
            zbuf.at[pl.ds(i, 1), pl.ds(0, DH)][...] = jnp.zeros((1, DH), jnp.float32)

        @pl.loop(0, CHUNK)
        def _(i):
            ones_v.at[pl.ds(i, 1), pl.ds(0, DH)][...] = jnp.ones((1, DH), jnp.float32)

        pltpu.sync_copy(dst_hbm.at[pl.ds(wid * CH_T, CH_T)], dstv)
        pltpu.sync_copy(zbuf, acc.at[pl.ds(sid * ZR, ZR)])
        plsc.subcore_barrier()

        @pl.loop(0, CH_T)
        def _(j):
            pltpu.sync_copy(ones_v, acc.at[dstv.at[j]], add=True)

        plsc.subcore_barrier()
        pltpu.sync_copy(
            acc.at[pl.ds(sid * ZR, ZR)],
            out_hbm.at[pl.ds(cid * NACC + sid * ZR, ZR)],
        )

    return k(dst2d)


def _sc_agg(src2d, dst2d, y):
    @functools.partial(
        pl.kernel,
        out_type=jax.ShapeDtypeStruct((NC * NACC, DH), jnp.float32),
        mesh=_MESH,
        compiler_params=_SC_PARAMS,
        scratch_types=[
            pltpu.VMEM((CH_T, CHUNK), jnp.int32),
            pltpu.VMEM((CH_T, CHUNK), jnp.int32),
            pltpu.VMEM((CHUNK, DH), jnp.float32),
            pltpu.VMEM((ZR, DH), jnp.float32),
            pltpu.VMEM_SHARED((NACC, DH), jnp.float32),
        ],
    )
    def k(src_hbm, dst_hbm, y_hbm, out_hbm, srcv, dstv, rows, zbuf, acc):
        cid = lax.axis_index("c")
        sid = lax.axis_index("s")
        wid = sid * NC + cid

        @pl.loop(0, ZR)
        def _(i):
            zbuf.at[pl.ds(i, 1), pl.ds(0, DH)][...] = jnp.zeros((1, DH), jnp.float32)

        pltpu.sync_copy(src_hbm.at[pl.ds(wid * CH_T, CH_T)], srcv)
        pltpu.sync_copy(dst_hbm.at[pl.ds(wid * CH_T, CH_T)], dstv)
        pltpu.sync_copy(zbuf, acc.at[pl.ds(sid * ZR, ZR)])
        plsc.subcore_barrier()

        @pl.loop(0, CH_T)
        def _(j):
            pltpu.sync_copy(y_hbm.at[srcv.at[j]], rows)
            pltpu.sync_copy(rows, acc.at[dstv.at[j]], add=True)

        plsc.subcore_barrier()
        pltpu.sync_copy(
            acc.at[pl.ds(sid * ZR, ZR)],
            out_hbm.at[pl.ds(cid * NACC + sid * ZR, ZR)],
        )

    return k(src2d, dst2d, y)


def _tc_first(xp, W1, histp):
    def body(h_ref, x_ref, w_ref, y_ref, d_ref):
        hp = h_ref[...]
        cnt = hp[:NACC, 0:1] + hp[NACC:, 0:1]
        dinv = lax.rsqrt(cnt + 1.0)
        xw = jnp.dot(x_ref[...], w_ref[...],
                     preferred_element_type=jnp.float32,
                     precision=lax.Precision.HIGHEST)
        y_ref[...] = xw * dinv
        d_ref[...] = dinv

    return pl.pallas_call(
        body,
        out_shape=[
            jax.ShapeDtypeStruct((NACC, DH), jnp.float32),
            jax.ShapeDtypeStruct((NACC, 1), jnp.float32),
        ],
    )(histp, xp, W1)


def _tc_mid(aggp, y1, dinv, W2, b1):
    def body(a_ref, y_ref, d_ref, w_ref, b_ref, o_ref):
        ap = a_ref[...]
        z = (ap[:NACC] + ap[NACC:] + y_ref[...]) * d_ref[...] + b_ref[...]
        h = jnp.maximum(z, 0.0)
        o_ref[...] = jnp.dot(h, w_ref[...],
                             preferred_element_type=jnp.float32,
                             precision=lax.Precision.HIGHEST) * d_ref[...]

    return pl.pallas_call(
        body,
        out_shape=jax.ShapeDtypeStruct((NACC, DH), jnp.float32),
    )(aggp, y1, dinv, W2, b1)


def _tc_last(aggp, y2, dinv, b2):
    def body(a_ref, y_ref, d_ref, b_ref, o_ref):
        ap = a_ref[...]
        z = (ap[:NACC] + ap[NACC:] + y_ref[...]) * d_ref[...] + b_ref[...]
        m = jnp.max(z, axis=1, keepdims=True)
        s = z - m
        lse = jnp.log(jnp.sum(jnp.exp(s), axis=1, keepdims=True))
        o_ref[...] = s - lse

    return pl.pallas_call(
        body,
        out_shape=jax.ShapeDtypeStruct((NACC, DH), jnp.float32),
    )(aggp, y2, dinv, b2)


def kernel(x, edge_index, W1, b1, W2, b2):
    src = edge_index[0]
    dst = edge_index[1]
    pad = jnp.full((EP - E,), N, jnp.int32)
    src2d = jnp.concatenate([src, pad]).reshape(NW * CH_T, CHUNK)
    dst2d = jnp.concatenate([dst, pad]).reshape(NW * CH_T, CHUNK)
    xp = jnp.zeros((NACC, D_IN), jnp.float32).at[:N].set(x)

    histp = _sc_hist(dst2d)
    y1, dinv = _tc_first(xp, W1, histp)
    agg1 = _sc_agg(src2d, dst2d, y1)
    y2 = _tc_mid(agg1, y1, dinv, W2, b1.reshape(1, DH))
    agg2 = _sc_agg(src2d, dst2d, y2)
    out = _tc_last(agg2, y2, dinv, b2.reshape(1, DH))
    return out[:N]


# speedup vs baseline: 36.0733x; 1.2978x over previous
import functools

import jax
import jax.numpy as jnp
from jax import lax
from jax.experimental import pallas as pl
from jax.experimental.pallas import tpu as pltpu
from jax.experimental.pallas import tpu_sc as plsc

N = 10000
E = 320000
D_IN = 128
DH = 16
NC = 2
NS = 16
NW = NC * NS
CHUNK = 128
CH_T = 80
EP = NW * CH_T * CHUNK
NACC = 10112
ZR = NACC // NS
NB = 4

_MESH = plsc.VectorSubcoreMesh(core_axis_name="c", subcore_axis_name="s")
_SC_PARAMS = pltpu.CompilerParams(use_tc_tiling_on_sc=False)


def _sc_hist(dst2d):
    @functools.partial(
        pl.kernel,
        out_type=jax.ShapeDtypeStruct((NC * NACC, DH), jnp.float32),
        mesh=_MESH,
        compiler_params=_SC_PARAMS,
        scratch_types=[
            pltpu.VMEM((CH_T, CHUNK), jnp.int32),
            pltpu.VMEM((CHUNK, DH), jnp.float32),
            pltpu.VMEM((ZR, DH), jnp.float32),
            pltpu.VMEM_SHARED((NACC, DH), jnp.float32),
            pltpu.SemaphoreType.DMA,
        ],
    )
    def k(dst_hbm, out_hbm, dstv, ones_v, zbuf, acc, hsem):
        cid = lax.axis_index("c")
        sid = lax.axis_index("s")
        wid = sid * NC + cid

        @pl.loop(0, ZR)
        def _(i):
            zbuf.at[pl.ds(i, 1), pl.ds(0, DH)][...] = jnp.zeros((1, DH), jnp.float32)

        @pl.loop(0, CHUNK)
        def _(i):
            ones_v.at[pl.ds(i, 1), pl.ds(0, DH)][...] = jnp.ones((1, DH), jnp.float32)

        pltpu.sync_copy(dst_hbm.at[pl.ds(wid * CH_T, CH_T)], dstv)
        pltpu.sync_copy(zbuf, acc.at[pl.ds(sid * ZR, ZR)])
        plsc.subcore_barrier()

        @pl.loop(0, CH_T)
        def _(j):
            pltpu.async_copy(ones_v, acc.at[dstv.at[j]], hsem, add=True)

        @pl.loop(0, CH_T)
        def _(j):
            pltpu.make_async_copy(ones_v, acc.at[dstv.at[j]], hsem).wait()

        plsc.subcore_barrier()
        pltpu.sync_copy(
            acc.at[pl.ds(sid * ZR, ZR)],
            out_hbm.at[pl.ds(cid * NACC + sid * ZR, ZR)],
        )

    return k(dst2d)


def _sc_agg(src2d, dst2d, y):
    @functools.partial(
        pl.kernel,
        out_type=jax.ShapeDtypeStruct((NC * NACC, DH), jnp.float32),
        mesh=_MESH,
        compiler_params=_SC_PARAMS,
        scratch_types=[
            pltpu.VMEM((CH_T, CHUNK), jnp.int32),
            pltpu.VMEM((CH_T, CHUNK), jnp.int32),
            [pltpu.VMEM((CHUNK, DH), jnp.float32)] * NB,
            pltpu.VMEM((ZR, DH), jnp.float32),
            pltpu.VMEM_SHARED((NACC, DH), jnp.float32),
            [pltpu.SemaphoreType.DMA] * NB,
            [pltpu.SemaphoreType.DMA] * NB,
        ],
    )
    def k(src_hbm, dst_hbm, y_hbm, out_hbm, srcv, dstv, rows, zbuf, acc, gs, ss):
        cid = lax.axis_index("c")
        sid = lax.axis_index("s")
        wid = sid * NC + cid

        @pl.loop(0, ZR)
        def _(i):
            zbuf.at[pl.ds(i, 1), pl.ds(0, DH)][...] = jnp.zeros((1, DH), jnp.float32)

        pltpu.sync_copy(src_hbm.at[pl.ds(wid * CH_T, CH_T)], srcv)
        pltpu.sync_copy(dst_hbm.at[pl.ds(wid * CH_T, CH_T)], dstv)
        pltpu.sync_copy(zbuf, acc.at[pl.ds(sid * ZR, ZR)])
        plsc.subcore_barrier()

        for b in range(NB):
            pltpu.async_copy(y_hbm.at[srcv.at[b]], rows[b], gs[b])

        @pl.loop(0, CH_T - NB, step=NB)
        def _(j):
            for b in range(NB):
                pltpu.make_async_copy(y_hbm.at[srcv.at[j + b]], rows[b], gs[b]).wait()
                pltpu.async_copy(rows[b], acc.at[dstv.at[j + b]], ss[b], add=True)
            for b in range(NB):
                pltpu.make_async_copy(rows[b], acc.at[dstv.at[j + b]], ss[b]).wait()
                pltpu.async_copy(y_hbm.at[srcv.at[j + NB + b]], rows[b], gs[b])

        j0 = CH_T - NB
        for b in range(NB):
            pltpu.make_async_copy(y_hbm.at[srcv.at[j0 + b]], rows[b], gs[b]).wait()
            pltpu.async_copy(rows[b], acc.at[dstv.at[j0 + b]], ss[b], add=True)
        for b in range(NB):
            pltpu.make_async_copy(rows[b], acc.at[dstv.at[j0 + b]], ss[b]).wait()

        plsc.subcore_barrier()
        pltpu.sync_copy(
            acc.at[pl.ds(sid * ZR, ZR)],
            out_hbm.at[pl.ds(cid * NACC + sid * ZR, ZR)],
        )

    return k(src2d, dst2d, y)


def _tc_first(xp, W1, histp):
    def body(h_ref, x_ref, w_ref, y_ref, d_ref):
        hp = h_ref[...]
        cnt = hp[:NACC, 0:1] + hp[NACC:, 0:1]
        dinv = lax.rsqrt(cnt + 1.0)
        xw = jnp.dot(x_ref[...], w_ref[...],
                     preferred_element_type=jnp.float32,
                     precision=lax.Precision.HIGHEST)
        y_ref[...] = xw * dinv
        d_ref[...] = dinv

    return pl.pallas_call(
        body,
        out_shape=[
            jax.ShapeDtypeStruct((NACC, DH), jnp.float32),
            jax.ShapeDtypeStruct((NACC, 1), jnp.float32),
        ],
    )(histp, xp, W1)


def _tc_mid(aggp, y1, dinv, W2, b1):
    def body(a_ref, y_ref, d_ref, w_ref, b_ref, o_ref):
        ap = a_ref[...]
        z = (ap[:NACC] + ap[NACC:] + y_ref[...]) * d_ref[...] + b_ref[...]
        h = jnp.maximum(z, 0.0)
        o_ref[...] = jnp.dot(h, w_ref[...],
                             preferred_element_type=jnp.float32,
                             precision=lax.Precision.HIGHEST) * d_ref[...]

    return pl.pallas_call(
        body,
        out_shape=jax.ShapeDtypeStruct((NACC, DH), jnp.float32),
    )(aggp, y1, dinv, W2, b1)


def _tc_last(aggp, y2, dinv, b2):
    def body(a_ref, y_ref, d_ref, b_ref, o_ref):
        ap = a_ref[...]
        z = (ap[:NACC] + ap[NACC:] + y_ref[...]) * d_ref[...] + b_ref[...]
        m = jnp.max(z, axis=1, keepdims=True)
        s = z - m
        lse = jnp.log(jnp.sum(jnp.exp(s), axis=1, keepdims=True))
        o_ref[...] = s - lse

    return pl.pallas_call(
        body,
        out_shape=jax.ShapeDtypeStruct((NACC, DH), jnp.float32),
    )(aggp, y2, dinv, b2)


def kernel(x, edge_index, W1, b1, W2, b2):
    src = edge_index[0]
    dst = edge_index[1]
    pad = jnp.full((EP - E,), N, jnp.int32)
    src2d = jnp.concatenate([src, pad]).reshape(NW * CH_T, CHUNK)
    dst2d = jnp.concatenate([dst, pad]).reshape(NW * CH_T, CHUNK)
    xp = jnp.zeros((NACC, D_IN), jnp.float32).at[:N].set(x)

    histp = _sc_hist(dst2d)
    y1, dinv = _tc_first(xp, W1, histp)
    agg1 = _sc_agg(src2d, dst2d, y1)
    y2 = _tc_mid(agg1, y1, dinv, W2, b1.reshape(1, DH))
    agg2 = _sc_agg(src2d, dst2d, y2)
    out = _tc_last(agg2, y2, dinv, b2.reshape(1, DH))
    return out[:N]
